# baseline (speedup 1.0000x reference)
import functools

import jax
import jax.numpy as jnp
import numpy as np
from jax import lax
from jax.experimental import pallas as pl
from jax.experimental.pallas import tpu as pltpu
from jax.experimental.pallas import tpu_sc as plsc

_NUM_CORES = 2
_NUM_TILES = 16
_C = 40
_WR = 40


_QS = 8192.0
_QINV = 1.0 / _QS


def _interleave_perm(d):
    h = d // 2
    perm = np.empty(d, dtype=np.int32)
    for k in range(h):
        perm[k] = 32 * (k // 16) + k % 16
        perm[h + k] = 32 * (k // 16) + 16 + k % 16
    return perm


def _mm2_body(x_ref, w_ref, o_ref):
    o_ref[0] = jnp.dot(x_ref[0], w_ref[0], preferred_element_type=jnp.float32)


def _mm2_pack(x2, w2, bm):
    _, m, k = x2.shape
    d = w2.shape[2]
    h = d // 2
    wlo = w2[..., :h]
    whi = w2[..., h:]

    def body(x_ref, wlo_ref, whi_ref, o_ref):
        def q(w_ref):
            r = jnp.dot(
                x_ref[0], w_ref[0], preferred_element_type=jnp.float32)
            return jnp.clip(
                jnp.round(r * _QS), -32768.0, 32767.0).astype(jnp.int32)

        o_ref[0] = jnp.bitwise_or(
            jnp.bitwise_and(q(wlo_ref), 0xFFFF),
            lax.shift_left(q(whi_ref), 16))

    return pl.pallas_call(
        body,
        grid=(2, m // bm),
        in_specs=[
            pl.BlockSpec((1, bm, k), lambda b, i: (b, i, 0)),
            pl.BlockSpec((1, k, h), lambda b, i: (b, 0, 0)),
            pl.BlockSpec((1, k, h), lambda b, i: (b, 0, 0)),
        ],
        out_specs=pl.BlockSpec((1, bm, h), lambda b, i: (b, i, 0)),
        out_shape=jax.ShapeDtypeStruct((2, m, h), jnp.int32),
    )(x2, wlo, whi)


def _mm2(x2, w2, bm):
    _, m, k = x2.shape
    d = w2.shape[2]
    return pl.pallas_call(
        _mm2_body,
        grid=(2, m // bm),
        in_specs=[
            pl.BlockSpec((1, bm, k), lambda b, i: (b, i, 0)),
            pl.BlockSpec((1, k, d), lambda b, i: (b, 0, 0)),
        ],
        out_specs=pl.BlockSpec((1, bm, d), lambda b, i: (b, i, 0)),
        out_shape=jax.ShapeDtypeStruct((2, m, d), jnp.float32),
    )(x2, w2)


def _combine_body(x_ref, w_ref, a_ref, o_ref):
    o_ref[0] = jnp.maximum(
        jnp.dot(x_ref[0], w_ref[0], preferred_element_type=jnp.float32)
        + a_ref[0],
        0.0,
    )


def _combine(x2, w2, agg2, bm):
    _, n, d = x2.shape
    return pl.pallas_call(
        _combine_body,
        grid=(2, n // bm),
        in_specs=[
            pl.BlockSpec((1, bm, d), lambda b, i: (b, i, 0)),
            pl.BlockSpec((1, d, d), lambda b, i: (b, 0, 0)),
            pl.BlockSpec((1, bm, d), lambda b, i: (1 - b, i, 0)),
        ],
        out_specs=pl.BlockSpec((1, bm, d), lambda b, i: (b, i, 0)),
        out_shape=jax.ShapeDtypeStruct((2, n, d), jnp.float32),
    )(x2, w2, agg2)


def _make_sc_edge(n, d, e):
    ept = e // _NUM_TILES
    nchunks = ept // _C
    rchunks = n // _WR
    maxk = -(-rchunks // _NUM_TILES)
    mesh = plsc.VectorSubcoreMesh(
        core_axis_name="c", subcore_axis_name="s",
        num_cores=_NUM_CORES, num_subcores=_NUM_TILES,
    )

    @functools.partial(
        pl.kernel,
        out_type=jax.ShapeDtypeStruct((2 * n, d), jnp.float32),
        mesh=mesh,
        scratch_types=[
            pltpu.VMEM((ept,), jnp.int32),
            pltpu.VMEM((_C,), jnp.int32),
            pltpu.VMEM((_C,), jnp.int32),
            pltpu.VMEM((_C, d), jnp.float32),
            pltpu.VMEM((_C, d), jnp.float32),
            pltpu.VMEM((_C * d // 2,), jnp.int32),
            pltpu.VMEM((_C * d // 2,), jnp.int32),
            pltpu.VMEM_SHARED((n, d), jnp.float32),
            pltpu.SemaphoreType.DMA,
            pltpu.SemaphoreType.DMA,
            pltpu.SemaphoreType.DMA,
            pltpu.SemaphoreType.DMA,
        ],
    )
    def sc_edge(a_hbm, b_hbm, src_hbm, dst_hbm, out_hbm,
                src_v, idx_d0, idx_d1, rows0, rows1, bbuf0, bbuf1,
                agg, sem_in0, sem_in1, sem_sc0, sem_sc1):
        zbuf = rows0
        c = lax.axis_index("c")
        s = lax.axis_index("s")
        idx_d = (idx_d0, idx_d1)
        rows = (rows0, rows1)
        bbuf = (bbuf0, bbuf1)
        sem_in = (sem_in0, sem_in1)
        sem_sc = (sem_sc0, sem_sc1)

        ebase = c * e + s * ept
        coff = c * n

        src_stage = pltpu.async_copy(src_hbm.at[pl.ds(ebase, ept)], src_v,
                                     sem_in0)

        @pl.loop(0, _WR)
        def _zero(r):
            for j in range(d // 16):
                zbuf[r, pl.ds(j * 16, 16)] = jnp.zeros((16,), jnp.float32)

        for k in range(maxk):
            rc = s + k * _NUM_TILES
            @pl.when(rc < rchunks)
            def _():
                pltpu.sync_copy(zbuf, agg.at[pl.ds(rc * _WR, _WR)])
        src_stage.wait()
        plsc.subcore_barrier()

        @pl.loop(0, nchunks, step=2)
        def _pair(i):
            descs = []
            for b in range(2):
                @pl.when(i >= 2)
                def _():
                    pltpu.make_async_copy(
                        a_hbm.at[pl.ds(0, _C)], rows[b], sem_sc[b]).wait()
                off = ebase + (i + b) * _C
                g = pltpu.async_copy(
                    a_hbm.at[src_v.at[pl.ds((i + b) * _C, _C)]],
                    rows[b], sem_in[b])
                bb = pltpu.async_copy(
                    b_hbm.at[pl.ds(off * (d // 2), _C * (d // 2))],
                    bbuf[b], sem_in[b])
                di = pltpu.async_copy(dst_hbm.at[pl.ds(off, _C)], idx_d[b],
                                      sem_in[b])
                descs.append((g, bb, di))

            for b in range(2):
                for dsc in descs[b]:
                    dsc.wait()

                @plsc.parallel_loop(0, _C, unroll=2)
                def _edge(ei):
                    for j in range(d // 32):
                        w = bbuf[b][pl.ds(ei * (d // 2) + j * 16, 16)]
                        lo = lax.shift_right_arithmetic(
                            lax.shift_left(w, 16), 16).astype(jnp.float32) * _QINV
                        hi = lax.shift_right_arithmetic(
                            w, 16).astype(jnp.float32) * _QINV
                        sl0 = pl.ds(j * 32, 16)
                        sl1 = pl.ds(j * 32 + 16, 16)
                        rows[b][ei, sl0] = jnp.maximum(
                            rows[b][ei, sl0] + lo, 0.0)
                        rows[b][ei, sl1] = jnp.maximum(
                            rows[b][ei, sl1] + hi, 0.0)

                pltpu.async_copy(rows[b], agg.at[idx_d[b]], sem_sc[b],
                                 add=True)

        for b in range(2):
            pltpu.make_async_copy(
                a_hbm.at[pl.ds(0, _C)], rows[b], sem_sc[b]).wait()
        plsc.subcore_barrier()

        for k in range(maxk):
            rc = s + k * _NUM_TILES
            @pl.when(rc < rchunks)
            def _():
                pltpu.sync_copy(agg.at[pl.ds(rc * _WR, _WR)], zbuf)
                pltpu.sync_copy(zbuf, out_hbm.at[pl.ds(coff + rc * _WR, _WR)])

    return sc_edge


def kernel(x_adm, x_item, edge_index_fwd, edge_index_rev,
           edge_attr_fwd, edge_attr_rev, Wmsg, Wedge, Wself):
    t_steps, n, d = x_adm.shape
    e = edge_index_fwd.shape[2]
    n_layers = Wmsg.shape[1]
    sc_edge = _make_sc_edge(n, d, e)

    outs_a, outs_i = [], []
    for t in range(t_steps):
        x2 = jnp.stack([x_adm[t], x_item[t]])
        ea2 = jnp.stack([edge_attr_fwd[t], edge_attr_rev[t]])
        src = jnp.concatenate([edge_index_fwd[t, 0], edge_index_rev[t, 0] + n])
        dst = jnp.concatenate([edge_index_fwd[t, 1], edge_index_rev[t, 1]])
        perm = _interleave_perm(d)
        for l in range(n_layers):
            a2 = _mm2(x2, Wmsg[t, l], 400)
            b2 = _mm2_pack(ea2, Wedge[t, l][..., perm], 1600)
            agg = sc_edge(a2.reshape(2 * n, d), b2.reshape(-1), src, dst)
            x2 = _combine(x2, Wself[t, l], agg.reshape(2, n, d), 400)
        outs_a.append(x2[0])
        outs_i.append(x2[1])
    return jnp.stack([jnp.stack(outs_a), jnp.stack(outs_i)])


# speedup vs baseline: 1.2719x; 1.2719x over previous
import functools

import jax
import jax.numpy as jnp
import numpy as np
from jax import lax
from jax.experimental import pallas as pl
from jax.experimental.pallas import tpu as pltpu
from jax.experimental.pallas import tpu_sc as plsc

_NUM_CORES = 2
_NUM_TILES = 16
_C = 40
_WR = 40


_QS = 8192.0
_QINV = 1.0 / _QS


def _interleave_perm(d):
    h = d // 2
    perm = np.empty(d, dtype=np.int32)
    for k in range(h):
        perm[k] = 32 * (k // 16) + k % 16
        perm[h + k] = 32 * (k // 16) + 16 + k % 16
    return perm


def _mm2_body(x_ref, w_ref, o_ref):
    o_ref[0] = jnp.dot(x_ref[0], w_ref[0], preferred_element_type=jnp.float32)


def _mm2_pack(x2, w2, bm):
    _, m, k = x2.shape
    d = w2.shape[2]
    h = d // 2
    wlo = w2[..., :h]
    whi = w2[..., h:]

    def body(x_ref, wlo_ref, whi_ref, o_ref):
        def q(w_ref):
            r = jnp.dot(
                x_ref[0], w_ref[0], preferred_element_type=jnp.float32)
            return jnp.clip(
                jnp.round(r * _QS), -32768.0, 32767.0).astype(jnp.int32)

        o_ref[0] = jnp.bitwise_or(
            jnp.bitwise_and(q(wlo_ref), 0xFFFF),
            lax.shift_left(q(whi_ref), 16))

    return pl.pallas_call(
        body,
        grid=(2, m // bm),
        in_specs=[
            pl.BlockSpec((1, bm, k), lambda b, i: (b, i, 0)),
            pl.BlockSpec((1, k, h), lambda b, i: (b, 0, 0)),
            pl.BlockSpec((1, k, h), lambda b, i: (b, 0, 0)),
        ],
        out_specs=pl.BlockSpec((1, bm, h), lambda b, i: (b, i, 0)),
        out_shape=jax.ShapeDtypeStruct((2, m, h), jnp.int32),
    )(x2, wlo, whi)


def _mm2(x2, w2, bm):
    _, m, k = x2.shape
    d = w2.shape[2]
    return pl.pallas_call(
        _mm2_body,
        grid=(2, m // bm),
        in_specs=[
            pl.BlockSpec((1, bm, k), lambda b, i: (b, i, 0)),
            pl.BlockSpec((1, k, d), lambda b, i: (b, 0, 0)),
        ],
        out_specs=pl.BlockSpec((1, bm, d), lambda b, i: (b, i, 0)),
        out_shape=jax.ShapeDtypeStruct((2, m, d), jnp.float32),
    )(x2, w2)


def _combine_body(x_ref, w_ref, a_ref, o_ref):
    o_ref[0] = jnp.maximum(
        jnp.dot(x_ref[0], w_ref[0], preferred_element_type=jnp.float32)
        + a_ref[0],
        0.0,
    )


def _combine(x2, w2, agg2, bm):
    _, n, d = x2.shape
    return pl.pallas_call(
        _combine_body,
        grid=(2, n // bm),
        in_specs=[
            pl.BlockSpec((1, bm, d), lambda b, i: (b, i, 0)),
            pl.BlockSpec((1, d, d), lambda b, i: (b, 0, 0)),
            pl.BlockSpec((1, bm, d), lambda b, i: (1 - b, i, 0)),
        ],
        out_specs=pl.BlockSpec((1, bm, d), lambda b, i: (b, i, 0)),
        out_shape=jax.ShapeDtypeStruct((2, n, d), jnp.float32),
    )(x2, w2, agg2)


def _make_sc_edge(n, d, e):
    ept = e // _NUM_TILES
    nchunks = ept // _C
    rchunks = n // _WR
    maxk = -(-rchunks // _NUM_TILES)
    mesh = plsc.VectorSubcoreMesh(
        core_axis_name="c", subcore_axis_name="s",
        num_cores=_NUM_CORES, num_subcores=_NUM_TILES,
    )

    @functools.partial(
        pl.kernel,
        out_type=jax.ShapeDtypeStruct((2 * n, d), jnp.float32),
        mesh=mesh,
        scratch_types=[
            pltpu.VMEM((ept,), jnp.int32),
            pltpu.VMEM((_C,), jnp.int32),
            pltpu.VMEM((_C,), jnp.int32),
            pltpu.VMEM((_C, d), jnp.float32),
            pltpu.VMEM((_C, d), jnp.float32),
            pltpu.VMEM((_C, d), jnp.float32),
            pltpu.VMEM((_C, d), jnp.float32),
            pltpu.VMEM_SHARED((n, d), jnp.float32),
            pltpu.SemaphoreType.DMA,
            pltpu.SemaphoreType.DMA,
            pltpu.SemaphoreType.DMA,
            pltpu.SemaphoreType.DMA,
        ],
    )
    def sc_edge(a_hbm, b_hbm, src_hbm, dst_hbm, out_hbm,
                src_v, idx_d0, idx_d1, rows0, rows1, bbuf0, bbuf1,
                agg, sem_in0, sem_in1, sem_sc0, sem_sc1):
        zbuf = rows0
        c = lax.axis_index("c")
        s = lax.axis_index("s")
        idx_d = (idx_d0, idx_d1)
        rows = (rows0, rows1)
        bbuf = (bbuf0, bbuf1)
        sem_in = (sem_in0, sem_in1)
        sem_sc = (sem_sc0, sem_sc1)

        ebase = c * e + s * ept
        coff = c * n

        src_stage = pltpu.async_copy(src_hbm.at[pl.ds(ebase, ept)], src_v,
                                     sem_in0)

        @pl.loop(0, _WR)
        def _zero(r):
            for j in range(d // 16):
                zbuf[r, pl.ds(j * 16, 16)] = jnp.zeros((16,), jnp.float32)

        for k in range(maxk):
            rc = s + k * _NUM_TILES
            @pl.when(rc < rchunks)
            def _():
                pltpu.sync_copy(zbuf, agg.at[pl.ds(rc * _WR, _WR)])
        src_stage.wait()
        plsc.subcore_barrier()

        @pl.loop(0, nchunks, step=2)
        def _pair(i):
            descs = []
            for b in range(2):
                @pl.when(i >= 2)
                def _():
                    pltpu.make_async_copy(
                        a_hbm.at[pl.ds(0, _C)], rows[b], sem_sc[b]).wait()
                off = ebase + (i + b) * _C
                g = pltpu.async_copy(
                    a_hbm.at[src_v.at[pl.ds((i + b) * _C, _C)]],
                    rows[b], sem_in[b])
                bb = pltpu.async_copy(b_hbm.at[pl.ds(off, _C)], bbuf[b],
                                      sem_in[b])
                di = pltpu.async_copy(dst_hbm.at[pl.ds(off, _C)], idx_d[b],
                                      sem_in[b])
                descs.append((g, bb, di))

            for b in range(2):
                for dsc in descs[b]:
                    dsc.wait()

                @plsc.parallel_loop(0, _C, unroll=4)
                def _edge(ei):
                    for j in range(d // 16):
                        sl = pl.ds(j * 16, 16)
                        rows[b][ei, sl] = jnp.maximum(
                            rows[b][ei, sl] + bbuf[b][ei, sl], 0.0)

                pltpu.async_copy(rows[b], agg.at[idx_d[b]], sem_sc[b],
                                 add=True)

        for b in range(2):
            pltpu.make_async_copy(
                a_hbm.at[pl.ds(0, _C)], rows[b], sem_sc[b]).wait()
        plsc.subcore_barrier()

        for k in range(maxk):
            rc = s + k * _NUM_TILES
            @pl.when(rc < rchunks)
            def _():
                pltpu.sync_copy(agg.at[pl.ds(rc * _WR, _WR)], zbuf)
                pltpu.sync_copy(zbuf, out_hbm.at[pl.ds(coff + rc * _WR, _WR)])

    return sc_edge


def kernel(x_adm, x_item, edge_index_fwd, edge_index_rev,
           edge_attr_fwd, edge_attr_rev, Wmsg, Wedge, Wself):
    t_steps, n, d = x_adm.shape
    e = edge_index_fwd.shape[2]
    n_layers = Wmsg.shape[1]
    sc_edge = _make_sc_edge(n, d, e)

    outs_a, outs_i = [], []
    for t in range(t_steps):
        x2 = jnp.stack([x_adm[t], x_item[t]])
        ea2 = jnp.stack([edge_attr_fwd[t], edge_attr_rev[t]])
        src = jnp.concatenate([edge_index_fwd[t, 0], edge_index_rev[t, 0] + n])
        dst = jnp.concatenate([edge_index_fwd[t, 1], edge_index_rev[t, 1]])
        perm = _interleave_perm(d)
        for l in range(n_layers):
            a2 = _mm2(x2, Wmsg[t, l], 400)
            b2 = _mm2(ea2, Wedge[t, l], 1600)
            agg = sc_edge(a2.reshape(2 * n, d), b2.reshape(2 * e, d), src, dst)
            x2 = _combine(x2, Wself[t, l], agg.reshape(2, n, d), 400)
        outs_a.append(x2[0])
        outs_i.append(x2[1])
    return jnp.stack([jnp.stack(outs_a), jnp.stack(outs_i)])


# speedup vs baseline: 1.3581x; 1.0678x over previous
import functools

import jax
import jax.numpy as jnp
import numpy as np
from jax import lax
from jax.experimental import pallas as pl
from jax.experimental.pallas import tpu as pltpu
from jax.experimental.pallas import tpu_sc as plsc

_NUM_CORES = 2
_NUM_TILES = 16
_C = 80
_WR = 80
_NSEG = 5


_QS = 8192.0
_QINV = 1.0 / _QS


def _interleave_perm(d):
    h = d // 2
    perm = np.empty(d, dtype=np.int32)
    for k in range(h):
        perm[k] = 32 * (k // 16) + k % 16
        perm[h + k] = 32 * (k // 16) + 16 + k % 16
    return perm


def _mm2_body(x_ref, w_ref, o_ref):
    o_ref[0] = jnp.dot(x_ref[0], w_ref[0], preferred_element_type=jnp.float32)


def _mm2_pack(x2, w2, bm):
    _, m, k = x2.shape
    d = w2.shape[2]
    h = d // 2
    wlo = w2[..., :h]
    whi = w2[..., h:]

    def body(x_ref, wlo_ref, whi_ref, o_ref):
        def q(w_ref):
            r = jnp.dot(
                x_ref[0], w_ref[0], preferred_element_type=jnp.float32)
            return jnp.clip(
                jnp.round(r * _QS), -32768.0, 32767.0).astype(jnp.int32)

        o_ref[0] = jnp.bitwise_or(
            jnp.bitwise_and(q(wlo_ref), 0xFFFF),
            lax.shift_left(q(whi_ref), 16))

    return pl.pallas_call(
        body,
        grid=(2, m // bm),
        in_specs=[
            pl.BlockSpec((1, bm, k), lambda b, i: (b, i, 0)),
            pl.BlockSpec((1, k, h), lambda b, i: (b, 0, 0)),
            pl.BlockSpec((1, k, h), lambda b, i: (b, 0, 0)),
        ],
        out_specs=pl.BlockSpec((1, bm, h), lambda b, i: (b, i, 0)),
        out_shape=jax.ShapeDtypeStruct((2, m, h), jnp.int32),
    )(x2, wlo, whi)


def _mm2(x2, w2, bm):
    _, m, k = x2.shape
    d = w2.shape[2]
    return pl.pallas_call(
        _mm2_body,
        grid=(2, m // bm),
        in_specs=[
            pl.BlockSpec((1, bm, k), lambda b, i: (b, i, 0)),
            pl.BlockSpec((1, k, d), lambda b, i: (b, 0, 0)),
        ],
        out_specs=pl.BlockSpec((1, bm, d), lambda b, i: (b, i, 0)),
        out_shape=jax.ShapeDtypeStruct((2, m, d), jnp.float32),
    )(x2, w2)


def _combine_body(x_ref, w_ref, a_ref, o_ref):
    o_ref[0] = jnp.maximum(
        jnp.dot(x_ref[0], w_ref[0], preferred_element_type=jnp.float32)
        + a_ref[0],
        0.0,
    )


def _combine(x2, w2, agg2, bm):
    _, n, d = x2.shape
    return pl.pallas_call(
        _combine_body,
        grid=(2, n // bm),
        in_specs=[
            pl.BlockSpec((1, bm, d), lambda b, i: (b, i, 0)),
            pl.BlockSpec((1, d, d), lambda b, i: (b, 0, 0)),
            pl.BlockSpec((1, bm, d), lambda b, i: (1 - b, i, 0)),
        ],
        out_specs=pl.BlockSpec((1, bm, d), lambda b, i: (b, i, 0)),
        out_shape=jax.ShapeDtypeStruct((2, n, d), jnp.float32),
    )(x2, w2, agg2)


def _make_sc_edge(n, d, e):
    ept = e // _NUM_TILES
    seg_edges = ept // _NSEG
    seg_chunks = seg_edges // _C
    rchunks = n // _WR
    maxk = -(-rchunks // _NUM_TILES)
    mesh = plsc.VectorSubcoreMesh(
        core_axis_name="c", subcore_axis_name="s",
        num_cores=_NUM_CORES, num_subcores=_NUM_TILES,
    )

    @functools.partial(
        pl.kernel,
        out_type=jax.ShapeDtypeStruct((2 * n, d), jnp.float32),
        mesh=mesh,
        scratch_types=[
            pltpu.VMEM((seg_edges,), jnp.int32),
            pltpu.VMEM((seg_edges,), jnp.int32),
            pltpu.VMEM((_C,), jnp.int32),
            pltpu.VMEM((_C,), jnp.int32),
            pltpu.VMEM((_C, d), jnp.float32),
            pltpu.VMEM((_C, d), jnp.float32),
            pltpu.VMEM((_C, d), jnp.float32),
            pltpu.VMEM((_C, d), jnp.float32),
            pltpu.VMEM_SHARED((n, d), jnp.float32),
            pltpu.SemaphoreType.DMA,
            pltpu.SemaphoreType.DMA,
            pltpu.SemaphoreType.DMA,
            pltpu.SemaphoreType.DMA,
            pltpu.SemaphoreType.DMA,
            pltpu.SemaphoreType.DMA,
        ],
    )
    def sc_edge(a_hbm, b_hbm, src_hbm, dst_hbm, out_hbm,
                src_v0, src_v1, idx_d0, idx_d1, rows0, rows1, bbuf0, bbuf1,
                agg, sem_in0, sem_in1, sem_sc0, sem_sc1, sem_sv0, sem_sv1):
        zbuf = rows0
        c = lax.axis_index("c")
        s = lax.axis_index("s")
        src_v = (src_v0, src_v1)
        idx_d = (idx_d0, idx_d1)
        rows = (rows0, rows1)
        bbuf = (bbuf0, bbuf1)
        sem_in = (sem_in0, sem_in1)
        sem_sc = (sem_sc0, sem_sc1)
        sem_sv = (sem_sv0, sem_sv1)

        ebase = c * e + s * ept
        coff = c * n

        def stage_src(sg):
            return pltpu.async_copy(
                src_hbm.at[pl.ds(ebase + sg * seg_edges, seg_edges)],
                src_v[sg % 2], sem_sv[sg % 2])

        def drain_scatter(b):
            pltpu.make_async_copy(
                a_hbm.at[pl.ds(0, _C)], rows[b], sem_sc[b]).wait()

        s0 = stage_src(0)

        @pl.loop(0, _WR)
        def _zero(r):
            for j in range(d // 16):
                zbuf[r, pl.ds(j * 16, 16)] = jnp.zeros((16,), jnp.float32)

        for k in range(maxk):
            rc = s + k * _NUM_TILES
            @pl.when(rc < rchunks)
            def _():
                pltpu.sync_copy(zbuf, agg.at[pl.ds(rc * _WR, _WR)])
        s0.wait()
        plsc.subcore_barrier()

        for sg in range(_NSEG):
            if sg + 1 < _NSEG:
                stage_src(sg + 1)
            if sg > 0:
                pltpu.make_async_copy(
                    src_hbm.at[pl.ds(ebase, seg_edges)], src_v[sg % 2],
                    sem_sv[sg % 2]).wait()
            sv = src_v[sg % 2]
            seg_base = ebase + sg * seg_edges

            @pl.loop(0, seg_chunks, step=2)
            def _pair(i, sg=sg, sv=sv, seg_base=seg_base):
                descs = []
                for b in range(2):
                    if sg == 0:
                        @pl.when(i >= 2)
                        def _():
                            drain_scatter(b)
                    else:
                        drain_scatter(b)
                    off = seg_base + (i + b) * _C
                    g = pltpu.async_copy(
                        a_hbm.at[sv.at[pl.ds((i + b) * _C, _C)]],
                        rows[b], sem_in[b])
                    bb = pltpu.async_copy(b_hbm.at[pl.ds(off, _C)], bbuf[b],
                                          sem_in[b])
                    di = pltpu.async_copy(dst_hbm.at[pl.ds(off, _C)],
                                          idx_d[b], sem_in[b])
                    descs.append((g, bb, di))

                for b in range(2):
                    for dsc in descs[b]:
                        dsc.wait()

                    @plsc.parallel_loop(0, _C, unroll=4)
                    def _edge(ei):
                        for j in range(d // 16):
                            sl = pl.ds(j * 16, 16)
                            rows[b][ei, sl] = jnp.maximum(
                                rows[b][ei, sl] + bbuf[b][ei, sl], 0.0)

                    pltpu.async_copy(rows[b], agg.at[idx_d[b]], sem_sc[b],
                                     add=True)

        for b in range(2):
            drain_scatter(b)
        plsc.subcore_barrier()

        for k in range(maxk):
            rc = s + k * _NUM_TILES
            @pl.when(rc < rchunks)
            def _():
                pltpu.sync_copy(agg.at[pl.ds(rc * _WR, _WR)], zbuf)
                pltpu.sync_copy(zbuf, out_hbm.at[pl.ds(coff + rc * _WR, _WR)])

    return sc_edge


def kernel(x_adm, x_item, edge_index_fwd, edge_index_rev,
           edge_attr_fwd, edge_attr_rev, Wmsg, Wedge, Wself):
    t_steps, n, d = x_adm.shape
    e = edge_index_fwd.shape[2]
    n_layers = Wmsg.shape[1]
    sc_edge = _make_sc_edge(n, d, e)

    outs_a, outs_i = [], []
    for t in range(t_steps):
        x2 = jnp.stack([x_adm[t], x_item[t]])
        ea2 = jnp.stack([edge_attr_fwd[t], edge_attr_rev[t]])
        src = jnp.concatenate([edge_index_fwd[t, 0], edge_index_rev[t, 0] + n])
        dst = jnp.concatenate([edge_index_fwd[t, 1], edge_index_rev[t, 1]])
        perm = _interleave_perm(d)
        for l in range(n_layers):
            a2 = _mm2(x2, Wmsg[t, l], 400)
            b2 = _mm2(ea2, Wedge[t, l], 1600)
            agg = sc_edge(a2.reshape(2 * n, d), b2.reshape(2 * e, d), src, dst)
            x2 = _combine(x2, Wself[t, l], agg.reshape(2, n, d), 400)
        outs_a.append(x2[0])
        outs_i.append(x2[1])
    return jnp.stack([jnp.stack(outs_a), jnp.stack(outs_i)])
